# Initial kernel scaffold; baseline (speedup 1.0000x reference)
import jax
import jax.numpy as jnp
from jax.experimental import pallas as pl


def kernel(xyz, params):
    raise NotImplementedError("write your pallas kernel here")


# speedup vs baseline: 2.6824x; 2.6824x over previous
import functools

import jax
import jax.numpy as jnp
from jax.experimental import pallas as pl

F32 = jnp.float32


def _bn_pre(mean, var, g, be):
    c = mean.shape[-1]
    return (mean.reshape(1, c), g.reshape(1, c),
            jnp.sqrt(var + 1e-5).reshape(1, c), be.reshape(1, c))


def _mm(x, w, b, *, pre=None, relu=False, res=None, ln=None, rownorm=False,
        stats=False, bm=256):
    m, ci = x.shape
    co = w.shape[0]
    bm = min(bm, m)
    nblk = m // bm
    tiny = co < 8
    wx = w.T if tiny else w
    inputs = [x, wx, b.reshape(1, co)]
    specs = [pl.BlockSpec((bm, ci), lambda i: (i, 0)),
             pl.BlockSpec(wx.shape, lambda i: (0, 0)),
             pl.BlockSpec((1, co), lambda i: (0, 0))]
    if pre is not None:
        inputs += list(pre)
        specs += [pl.BlockSpec((1, ci), lambda i: (0, 0))] * 4
    if res is not None:
        inputs.append(res)
        specs.append(pl.BlockSpec((bm, co), lambda i: (i, 0)))
    if ln is not None:
        inputs += [ln[0].reshape(1, co), ln[1].reshape(1, co)]
        specs += [pl.BlockSpec((1, co), lambda i: (0, 0))] * 2
    out_shape = [jax.ShapeDtypeStruct((m, co), F32)]
    out_specs = [pl.BlockSpec((bm, co), lambda i: (i, 0))]
    if stats:
        out_shape.append(jax.ShapeDtypeStruct((8, co), F32))
        out_specs.append(pl.BlockSpec((8, co), lambda i: (0, 0)))

    def body(*refs):
        it = iter(refs)
        x_ref = next(it)
        wt_ref = next(it)
        b_ref = next(it)
        r_ref = g_ref = be_ref = None
        if pre is not None:
            pre_refs = [next(it) for _ in range(4)]
        if res is not None:
            r_ref = next(it)
        if ln is not None:
            g_ref = next(it)
            be_ref = next(it)
        y_ref = next(it)
        ss_ref = next(it) if stats else None
        xb = x_ref[...]
        if pre is not None:
            pm, pg, psq, pbe = [r[...] for r in pre_refs]
            xb = jnp.maximum(pg * (xb - pm) / psq + pbe, 0.0)
        if tiny:
            y = jnp.dot(xb, wt_ref[...],
                        preferred_element_type=F32) + b_ref[...]
        else:
            y = jax.lax.dot_general(
                xb, wt_ref[...], (((1,), (1,)), ((), ())),
                preferred_element_type=F32) + b_ref[...]
        if relu:
            y = jnp.maximum(y, 0.0)
        if res is not None:
            y = y + r_ref[...]
        if ln is not None:
            mu = jnp.mean(y, axis=-1, keepdims=True)
            va = jnp.mean((y - mu) ** 2, axis=-1, keepdims=True)
            y = g_ref[...] * (y - mu) / jnp.sqrt(va + 1e-5) + be_ref[...]
        if rownorm:
            nr = jnp.sqrt(jnp.sum(y * y, axis=-1, keepdims=True))
            y = y / jnp.maximum(nr, 1e-12)
        y_ref[...] = y
        if stats:
            row = jax.lax.broadcasted_iota(jnp.int32, (8, co), 0)
            s1 = jnp.sum(y, axis=0, keepdims=True)
            s2 = jnp.sum(y * y, axis=0, keepdims=True)
            blk = jnp.where(row == 0, s1, 0.0) + jnp.where(row == 1, s2, 0.0)

            @pl.when(pl.program_id(0) == 0)
            def _():
                ss_ref[...] = blk

            @pl.when(pl.program_id(0) != 0)
            def _():
                ss_ref[...] = ss_ref[...] + blk

    outs = pl.pallas_call(body, grid=(nblk,), in_specs=specs,
                          out_specs=out_specs, out_shape=out_shape)(*inputs)
    return outs if stats else outs[0]


def _fps(xyz_t, npoint):
    bsz, _, n = xyz_t.shape

    def body(x_ref, o_ref):
        xt = x_ref[0]
        lane = jax.lax.broadcasted_iota(jnp.int32, (1, n), 1)
        lane_np = jax.lax.broadcasted_iota(jnp.int32, (1, npoint), 1)

        def step(i, st):
            cent, dist, far = st
            cent = jnp.where(lane_np == i, far, cent)
            oh = (lane == far).astype(F32)
            c = jnp.sum(xt * oh, axis=1, keepdims=True)
            d = jnp.sum((xt - c) ** 2, axis=0, keepdims=True)
            dist = jnp.minimum(dist, d)
            far = jnp.argmax(dist).astype(jnp.int32)
            return cent, dist, far

        cent0 = jnp.zeros((1, npoint), jnp.int32)
        dist0 = jnp.full((1, n), 1e10, F32)
        cent, _, _ = jax.lax.fori_loop(0, npoint, step,
                                       (cent0, dist0, jnp.int32(0)))
        o_ref[0] = cent

    out = pl.pallas_call(
        body, grid=(bsz,),
        in_specs=[pl.BlockSpec((1, 3, n), lambda b: (b, 0, 0))],
        out_specs=pl.BlockSpec((1, 1, npoint), lambda b: (b, 0, 0)),
        out_shape=jax.ShapeDtypeStruct((bsz, 1, npoint), jnp.int32))(xyz_t)
    return out.reshape(bsz, npoint)


def _gather3(coords_t, idx):
    bsz, _, n = coords_t.shape
    s = idx.shape[1]
    idx3 = idx.reshape(bsz, s, 1)

    def body(i_ref, ct_ref, o_ref):
        ic = i_ref[0]
        ct = ct_ref[0]
        lane = jax.lax.broadcasted_iota(jnp.int32, (s, n), 1)
        oh = (ic == lane).astype(F32)
        cols = [jnp.sum(oh * ct[c:c + 1, :], axis=1, keepdims=True)
                for c in range(3)]
        o_ref[0] = jnp.concatenate(cols, axis=1)

    return pl.pallas_call(
        body, grid=(bsz,),
        in_specs=[pl.BlockSpec((1, s, 1), lambda b: (b, 0, 0)),
                  pl.BlockSpec((1, 3, n), lambda b: (b, 0, 0))],
        out_specs=pl.BlockSpec((1, s, 3), lambda b: (b, 0, 0)),
        out_shape=jax.ShapeDtypeStruct((bsz, s, 3), F32))(idx3, coords_t)


def _ball_query(xyz, new_xyz, radius, k):
    bsz, n, _ = xyz.shape
    s = new_xyz.shape[1]
    r2 = radius * radius

    def body(x_ref, q_ref, o_ref):
        xt = x_ref[0]
        q = q_ref[0]
        d2 = -2.0 * jax.lax.dot_general(q, xt, (((1,), (1,)), ((), ())),
                                        preferred_element_type=F32)
        d2 = d2 + jnp.sum(q * q, axis=1, keepdims=True)
        d2 = d2 + jnp.sum(xt * xt, axis=1, keepdims=True).T
        mask = d2 <= r2
        lane = jax.lax.broadcasted_iota(jnp.int32, (s, n), 1).astype(F32)
        lane_k = jax.lax.broadcasted_iota(jnp.int32, (s, k), 1)
        val = jnp.where(mask, lane, float(n))
        out = jnp.zeros((s, k), F32)
        idx0 = None
        for kk in range(k):
            ik = jnp.min(val, axis=1, keepdims=True)
            if kk == 0:
                idx0 = ik
            sel = jnp.where(ik < float(n), ik, idx0)
            out = jnp.where(lane_k == kk, sel, out)
            val = jnp.where(lane == ik, float(n), val)
        o_ref[0] = out.astype(jnp.int32)

    return pl.pallas_call(
        body, grid=(bsz,),
        in_specs=[pl.BlockSpec((1, n, 3), lambda b: (b, 0, 0)),
                  pl.BlockSpec((1, s, 3), lambda b: (b, 0, 0))],
        out_specs=pl.BlockSpec((1, s, k), lambda b: (b, 0, 0)),
        out_shape=jax.ShapeDtypeStruct((bsz, s, k), jnp.int32))(xyz, new_xyz)


def _group_mm(gidx, pts, coords, new_xyz, w, b, k):
    bsz, s, _ = gidx.shape
    n = pts.shape[1]
    cp = pts.shape[2]
    co = w.shape[0]
    m2 = s * k
    rb = min(2048, m2)
    nblk = m2 // rb
    gidx_f = gidx.reshape(bsz, m2, 1)
    nxe = jnp.repeat(new_xyz, k, axis=1)

    def body(g_ref, p_ref, c_ref, nx_ref, w_ref, b_ref, y_ref,
             ss_ref):
        gi = jnp.minimum(g_ref[0], n - 1)
        lane = jax.lax.broadcasted_iota(jnp.int32, (rb, n), 1)
        oh = (gi == lane).astype(F32)
        hi = jax.lax.Precision.HIGHEST
        gp = jnp.dot(oh, p_ref[0], precision=hi,
                     preferred_element_type=F32)
        gx = jnp.dot(oh, c_ref[0], precision=hi,
                     preferred_element_type=F32) - nx_ref[0]
        gcat = jnp.concatenate([gp, gx], axis=1)
        y = jax.lax.dot_general(gcat, w_ref[...], (((1,), (1,)), ((), ())),
                                preferred_element_type=F32) + b_ref[...]
        y_ref[...] = y
        row = jax.lax.broadcasted_iota(jnp.int32, (8, co), 0)
        s1 = jnp.sum(y, axis=0, keepdims=True)
        s2 = jnp.sum(y * y, axis=0, keepdims=True)
        blk = jnp.where(row == 0, s1, 0.0) + jnp.where(row == 1, s2, 0.0)
        first = jnp.logical_and(pl.program_id(0) == 0, pl.program_id(1) == 0)

        @pl.when(first)
        def _():
            ss_ref[...] = blk

        @pl.when(jnp.logical_not(first))
        def _():
            ss_ref[...] = ss_ref[...] + blk

    y, ss = pl.pallas_call(
        body, grid=(bsz, nblk),
        in_specs=[pl.BlockSpec((1, rb, 1), lambda bb, i: (bb, i, 0)),
                  pl.BlockSpec((1, n, cp), lambda bb, i: (bb, 0, 0)),
                  pl.BlockSpec((1, n, 3), lambda bb, i: (bb, 0, 0)),
                  pl.BlockSpec((1, rb, 3), lambda bb, i: (bb, i, 0)),
                  pl.BlockSpec((co, cp + 3), lambda bb, i: (0, 0)),
                  pl.BlockSpec((1, co), lambda bb, i: (0, 0))],
        out_specs=[pl.BlockSpec((rb, co), lambda bb, i: (bb * nblk + i, 0)),
                   pl.BlockSpec((8, co), lambda bb, i: (0, 0))],
        out_shape=[jax.ShapeDtypeStruct((bsz * m2, co), F32),
                   jax.ShapeDtypeStruct((8, co), F32)])(
        gidx_f, pts, coords, nxe, w, b.reshape(1, co))
    return y, ss


def _bn_relu_max(y, bs, k, pre):
    c = y.shape[1]
    y3 = y.reshape(bs, k, c)
    bsb = min(64, bs)
    nblk = bs // bsb

    def body(y_ref, m_ref, g_ref, sq_ref, be_ref, o_ref):
        z = (g_ref[...] * (y_ref[...] - m_ref[...]) / sq_ref[...]
             + be_ref[...])
        o_ref[...] = jnp.max(jnp.maximum(z, 0.0), axis=1)

    return pl.pallas_call(
        body, grid=(nblk,),
        in_specs=[pl.BlockSpec((bsb, k, c), lambda i: (i, 0, 0))]
        + [pl.BlockSpec((1, 1, c), lambda i: (0, 0, 0))] * 4,
        out_specs=pl.BlockSpec((bsb, c), lambda i: (i, 0)),
        out_shape=jax.ShapeDtypeStruct((bs, c), F32))(
        y3, *[p.reshape(1, 1, c) for p in pre])


def _bn_relu(y, pre, bm=256):
    m, c = y.shape
    bm = min(bm, m)

    def body(y_ref, m_ref, g_ref, sq_ref, be_ref, o_ref):
        z = (g_ref[...] * (y_ref[...] - m_ref[...]) / sq_ref[...]
             + be_ref[...])
        o_ref[...] = jnp.maximum(z, 0.0)

    return pl.pallas_call(
        body, grid=(m // bm,),
        in_specs=[pl.BlockSpec((bm, c), lambda i: (i, 0))]
        + [pl.BlockSpec((1, c), lambda i: (0, 0))] * 4,
        out_specs=pl.BlockSpec((bm, c), lambda i: (i, 0)),
        out_shape=jax.ShapeDtypeStruct((m, c), F32))(y, *pre)


def _csumsq(y, mean, bm=512):
    m, c = y.shape
    bm = min(bm, m)

    def body(y_ref, m_ref, o_ref):
        z = y_ref[...] - m_ref[...]
        row = jax.lax.broadcasted_iota(jnp.int32, (8, c), 0)
        blk = jnp.where(row == 0, jnp.sum(z * z, axis=0, keepdims=True), 0.0)

        @pl.when(pl.program_id(0) == 0)
        def _():
            o_ref[...] = blk

        @pl.when(pl.program_id(0) != 0)
        def _():
            o_ref[...] = o_ref[...] + blk

    return pl.pallas_call(
        body, grid=(m // bm,),
        in_specs=[pl.BlockSpec((bm, c), lambda i: (i, 0)),
                  pl.BlockSpec((1, c), lambda i: (0, 0))],
        out_specs=pl.BlockSpec((8, c), lambda i: (0, 0)),
        out_shape=jax.ShapeDtypeStruct((8, c), F32))(y, mean)


def _bn_stats(y, ss, m_rows, g, be):
    mean = ss[0] / m_rows
    css = _csumsq(y, mean.reshape(1, -1))
    var = css[0] / m_rows
    return _bn_pre(mean, var, g, be)


def _fp_interp_mm(x1, x2, p2, pts1, w, b):
    bsz, n1, _ = x1.shape
    s2 = x2.shape[1]
    c2 = p2.shape[2]
    c1 = 0 if pts1 is None else pts1.shape[2]
    co = w.shape[0]
    bn = min(256, n1)
    nblk = n1 // bn

    inputs = [x1, x2, p2, w, b.reshape(1, co)]
    specs = [pl.BlockSpec((1, bn, 3), lambda bb, i: (bb, i, 0)),
             pl.BlockSpec((1, s2, 3), lambda bb, i: (bb, 0, 0)),
             pl.BlockSpec((1, s2, c2), lambda bb, i: (bb, 0, 0)),
             pl.BlockSpec((co, c1 + c2), lambda bb, i: (0, 0)),
             pl.BlockSpec((1, co), lambda bb, i: (0, 0))]
    if pts1 is not None:
        inputs += [pts1]
        specs += [pl.BlockSpec((1, bn, c1), lambda bb, i: (bb, i, 0))]

    def body(*refs):
        it = iter(refs)
        x1_ref = next(it)
        x2_ref = next(it)
        p2_ref = next(it)
        w_ref = next(it)
        b_ref = next(it)
        if pts1 is not None:
            p1_ref = next(it)
        y_ref = next(it)
        ss_ref = next(it)
        xb = x1_ref[0]
        x2b = x2_ref[0]
        d = -2.0 * jax.lax.dot_general(xb, x2b, (((1,), (1,)), ((), ())),
                                       preferred_element_type=F32)
        d = d + jnp.sum(xb * xb, axis=1, keepdims=True)
        d = d + jnp.sum(x2b * x2b, axis=1, keepdims=True).T
        lane = jax.lax.broadcasted_iota(jnp.int32, (bn, s2), 1).astype(F32)
        asum = jnp.zeros((bn, s2), F32)
        rsum = jnp.zeros((bn, 1), F32)
        for _ in range(3):
            mn = jnp.min(d, axis=1, keepdims=True)
            eq = d == mn
            ik = jnp.min(jnp.where(eq, lane, float(s2)), axis=1,
                         keepdims=True)
            first = lane == ik
            rec = 1.0 / (mn + 1e-8)
            asum = asum + jnp.where(first, rec, 0.0)
            rsum = rsum + rec
            d = jnp.where(first, 1e30, d)
        a = asum / rsum
        interp = jnp.dot(a, p2_ref[0], preferred_element_type=F32)
        if pts1 is not None:
            h = jnp.concatenate([p1_ref[0], interp], axis=1)
        else:
            h = interp
        y = jax.lax.dot_general(h, w_ref[...], (((1,), (1,)), ((), ())),
                                preferred_element_type=F32) + b_ref[...]
        y_ref[...] = y
        row = jax.lax.broadcasted_iota(jnp.int32, (8, co), 0)
        s1r = jnp.sum(y, axis=0, keepdims=True)
        s2r = jnp.sum(y * y, axis=0, keepdims=True)
        blk = jnp.where(row == 0, s1r, 0.0) + jnp.where(row == 1, s2r, 0.0)
        fst = jnp.logical_and(pl.program_id(0) == 0, pl.program_id(1) == 0)

        @pl.when(fst)
        def _():
            ss_ref[...] = blk

        @pl.when(jnp.logical_not(fst))
        def _():
            ss_ref[...] = ss_ref[...] + blk

    y, ss = pl.pallas_call(
        body, grid=(bsz, nblk),
        in_specs=specs,
        out_specs=[pl.BlockSpec((bn, co), lambda bb, i: (bb * nblk + i, 0)),
                   pl.BlockSpec((8, co), lambda bb, i: (0, 0))],
        out_shape=[jax.ShapeDtypeStruct((bsz * n1, co), F32),
                   jax.ShapeDtypeStruct((8, co), F32)])(*inputs)
    return y, ss


def _attention(q, k, v):
    bsz, h, l, dh = q.shape
    bq = min(512, l)
    nblk = l // bq
    scale = 1.0 / float(dh) ** 0.5

    def body(q_ref, kt_ref, v_ref, o_ref):
        qb = q_ref[0, 0]
        s = jax.lax.dot_general(qb, kt_ref[0, 0], (((1,), (1,)), ((), ())),
                                preferred_element_type=F32) * scale
        s = s - jnp.max(s, axis=-1, keepdims=True)
        p = jnp.exp(s)
        p = p / jnp.sum(p, axis=-1, keepdims=True)
        o_ref[0, 0] = jnp.dot(p, v_ref[0, 0], preferred_element_type=F32)

    return pl.pallas_call(
        body, grid=(bsz, h, nblk),
        in_specs=[pl.BlockSpec((1, 1, bq, dh), lambda b2, h2, i: (b2, h2, i, 0)),
                  pl.BlockSpec((1, 1, l, dh), lambda b2, h2, i: (b2, h2, 0, 0)),
                  pl.BlockSpec((1, 1, l, dh), lambda b2, h2, i: (b2, h2, 0, 0))],
        out_specs=pl.BlockSpec((1, 1, bq, dh), lambda b2, h2, i: (b2, h2, i, 0)),
        out_shape=jax.ShapeDtypeStruct((bsz, h, l, dh), F32))(q, k, v)


def _sa(p, npoint, radii, ks, coords, pts):
    bsz = coords.shape[0]
    coords_t = coords.transpose(0, 2, 1)
    fps_idx = _fps(coords_t, npoint)
    new_xyz = _gather3(coords_t, fps_idx)
    outs = []
    for bi, (r, k) in enumerate(zip(radii, ks)):
        layers = p['b%d' % bi]
        gidx = _ball_query(coords, new_xyz, r, k)
        l0 = layers[0]
        y, ss = _group_mm(gidx, pts, coords, new_xyz, l0['W'], l0['b'], k)
        m = bsz * npoint * k
        pre = _bn_stats(y, ss, m, l0['g'], l0['be'])
        for l in layers[1:]:
            y, ss = _mm(y, l['W'], l['b'], pre=pre, stats=True)
            pre = _bn_stats(y, ss, m, l['g'], l['be'])
        out = _bn_relu_max(y, bsz * npoint, k, pre)
        outs.append(out.reshape(bsz, npoint, -1))
    return new_xyz, jnp.concatenate(outs, axis=-1)


def _fp(p, coords1, coords2, pts1, pts2):
    bsz, n1, _ = coords1.shape
    l0 = p[0]
    y, ss = _fp_interp_mm(coords1, coords2, pts2, pts1, l0['W'], l0['b'])
    m = bsz * n1
    pre = _bn_stats(y, ss, m, l0['g'], l0['be'])
    for l in p[1:]:
        y, ss = _mm(y, l['W'], l['b'], pre=pre, stats=True)
        pre = _bn_stats(y, ss, m, l['g'], l['be'])
    return _bn_relu(y, pre).reshape(bsz, n1, -1)


def _encoder_layer(p, x, bsz, l):
    d = x.shape[1]
    h = 4
    dh = d // h
    qkv = _mm(x, p['Wqkv'], p['bqkv'])
    q, k, v = jnp.split(qkv, 3, axis=-1)
    rs = lambda t: t.reshape(bsz, l, h, dh).transpose(0, 2, 1, 3)
    q, k, v = rs(q), rs(k), rs(v)
    o = _attention(q, k, v)
    o = o.transpose(0, 2, 1, 3).reshape(bsz * l, d)
    x = _mm(o, p['Wo'], p['bo'], res=x, ln=(p['ln1_g'], p['ln1_b']))
    hdn = _mm(x, p['W1'], p['b1'], relu=True)
    return _mm(hdn, p['W2'], p['b2'], res=x, ln=(p['ln2_g'], p['ln2_b']))


def _head(hp, x, m, rownorm=False):
    h, ss = _mm(x, hp['c1']['W'], hp['c1']['b'], stats=True)
    pre = _bn_stats(h, ss, m, hp['c1']['g'], hp['c1']['be'])
    return _mm(h, hp['c2']['W'], hp['c2']['b'], pre=pre, rownorm=rownorm)


@jax.jit
def kernel(xyz, params):
    bsz, _, n = xyz.shape
    pts0 = xyz.transpose(0, 2, 1)
    coords0 = pts0[..., :3]

    l1_xyz, l1_pts = _sa(params['sa1'], 1024, [0.05, 0.1], [16, 32],
                         coords0, pts0)
    l2_xyz, l2_pts = _sa(params['sa2'], 256, [0.1, 0.2], [16, 32],
                         l1_xyz, l1_pts)
    l3_xyz, l3_pts = _sa(params['sa3'], 64, [0.2, 0.4], [16, 32],
                         l2_xyz, l2_pts)
    l4_xyz, l4_pts = _sa(params['sa4'], 16, [0.4, 0.8], [16, 32],
                         l3_xyz, l3_pts)

    l3_pts = _fp(params['fp4'], l3_xyz, l4_xyz, l3_pts, l4_pts)
    l2_pts = _fp(params['fp3'], l2_xyz, l3_xyz, l2_pts, l3_pts)
    l1_pts = _fp(params['fp2'], l1_xyz, l2_xyz, l1_pts, l2_pts)
    l0_pts = _fp(params['fp1'], coords0, l1_xyz, None, l1_pts)

    m = bsz * n
    pe = params['pos_enc']
    p1 = _mm(coords0.reshape(m, 3), pe[0]['W'], pe[0]['b'], relu=True)
    x = _mm(p1, pe[1]['W'], pe[1]['b'], res=l0_pts.reshape(m, -1))

    xt = x
    for lp in params['tr']:
        xt = _encoder_layer(lp, xt, bsz, n)
    mp = params['tr_mlp']
    hdn = _mm(xt, mp[0]['W'], mp[0]['b'], relu=True)
    x = _mm(hdn, mp[1]['W'], mp[1]['b'], res=x)

    path = _head(params['path_head'], x, m)
    kp = _head(params['kp_head'], x, m)
    dirn = _head(params['dir_head'], x, m, rownorm=True)
    path = path.reshape(bsz, n, 1).transpose(0, 2, 1)
    kp = kp.reshape(bsz, n, 1).transpose(0, 2, 1)
    dirn = dirn.reshape(bsz, n, 3).transpose(0, 2, 1)
    return (path, kp, dirn)
